# baseline (speedup 1.0000x reference)
import functools

import jax
import jax.numpy as jnp
from jax import lax
from jax.experimental import pallas as pl
from jax.experimental.pallas import tpu as pltpu
from jax.experimental.pallas import tpu_sc as plsc

_B, _TB, _H, _W = 16, 512, 64, 64
_C, _K = 64, 4
_PLANE = _H * _W
_NW = 32
_PPW = (_B * _C) // _NW
_PPC = 2
_NCHUNK = _PPW // _PPC
_THRESH = 2.0


def _threshold_chunk(rows_ref, out_ref):
    def body(j, _):
        off = j * 16
        for p in range(_PPC):
            r0 = rows_ref[4 * p + 0, pl.ds(off, 16)]
            r1 = rows_ref[4 * p + 1, pl.ds(off, 16)]
            r2 = rows_ref[4 * p + 2, pl.ds(off, 16)]
            r3 = rows_ref[4 * p + 3, pl.ds(off, 16)]
            s = ((r0 + r1) + r2) + r3
            out_ref[p, pl.ds(off, 16)] = jnp.where(
                s >= _THRESH, jnp.float32(1.0), jnp.float32(0.0))
        return 0
    lax.fori_loop(0, _PLANE // 16, body, 0)


def _sc_body(x_hbm, idx_hbm, out_hbm,
             idx_v, rows_a, rows_b, out_a, out_b,
             gsem_a, gsem_b, osem_a, osem_b):
    wid = lax.axis_index("s") * 2 + lax.axis_index("c")
    pltpu.sync_copy(idx_hbm.at[wid], idx_v)

    rows = [rows_a, rows_b]
    outs = [out_a, out_b]
    gsems = [gsem_a, gsem_b]
    osems = [osem_a, osem_b]
    ghandles = [None, None]
    ohandles = [None, None]

    ghandles[0] = pltpu.async_copy(x_hbm.at[idx_v.at[0]], rows[0], gsems[0])
    for t in range(_NCHUNK):
        cur = t & 1
        nxt = 1 - cur
        if t + 1 < _NCHUNK:
            ghandles[nxt] = pltpu.async_copy(
                x_hbm.at[idx_v.at[t + 1]], rows[nxt], gsems[nxt])
        ghandles[cur].wait()
        if ohandles[cur] is not None:
            ohandles[cur].wait()
        _threshold_chunk(rows[cur], outs[cur])
        ohandles[cur] = pltpu.async_copy(
            outs[cur], out_hbm.at[pl.ds(wid * _PPW + _PPC * t, _PPC)],
            osems[cur])
    ohandles[0].wait()
    ohandles[1].wait()


_sc_kernel = functools.partial(
    pl.kernel,
    out_type=jax.ShapeDtypeStruct((_B * _C, _PLANE), jnp.float32),
    mesh=plsc.VectorSubcoreMesh(core_axis_name="c", subcore_axis_name="s"),
    scratch_types=[
        pltpu.VMEM((_NCHUNK, 8), jnp.int32),
        pltpu.VMEM((4 * _PPC, _PLANE), jnp.float32),
        pltpu.VMEM((4 * _PPC, _PLANE), jnp.float32),
        pltpu.VMEM((_PPC, _PLANE), jnp.float32),
        pltpu.VMEM((_PPC, _PLANE), jnp.float32),
        pltpu.SemaphoreType.DMA,
        pltpu.SemaphoreType.DMA,
        pltpu.SemaphoreType.DMA,
        pltpu.SemaphoreType.DMA,
    ],
    compiler_params=pltpu.CompilerParams(use_tc_tiling_on_sc=True),
)(_sc_body)


def kernel(x, rp_map_idx):
    xf = x.reshape(_B * _TB, _PLANE)
    base = (jnp.arange(_B, dtype=jnp.int32) * _TB)[:, None, None]
    flat = (base + rp_map_idx[None, :, :].astype(jnp.int32))
    idx = flat.reshape(_NW, _NCHUNK, _PPC * _K)
    y = _sc_kernel(xf, idx)
    return y.reshape(_B, _C, _H, _W)


# speedup vs baseline: 1.1225x; 1.1225x over previous
import functools

import jax
import jax.numpy as jnp
from jax import lax
from jax.experimental import pallas as pl
from jax.experimental.pallas import tpu as pltpu
from jax.experimental.pallas import tpu_sc as plsc

_B, _TB, _H, _W = 16, 512, 64, 64
_C, _K = 64, 4
_NW = 32
_SLABS = 64
_THRESH = 2.0


def _compute_slab(slab, idxv, out_v, quarter):
    def cbody(c, _):
        accs = [None, None]
        for k in range(_K):
            ir = idxv[4 * c + k, :]
            ic = idxv[256 + 4 * c + k, :]
            for wc in range(2):
                irw = ir + jnp.int32(64 * wc) if wc else ir
                g = plsc.load_gather(slab, [irw, ic])
                accs[wc] = g if k == 0 else accs[wc] + g
        for wc in range(2):
            y = jnp.where(accs[wc] >= _THRESH, jnp.float32(1.0),
                          jnp.float32(0.0))
            out_v[c, pl.ds(quarter * 32 + wc * 16, 16)] = y
        return 0
    lax.fori_loop(0, _C, cbody, 0)


def _sc_body(x_hbm, tab_hbm, out_hbm,
             idxv, slab_a, slab_b, ov_a, ov_b,
             gs_a, gs_b, os_a, os_b):
    wid = lax.axis_index("s") * 2 + lax.axis_index("c")
    b = wid // 2
    qbase = (wid % 2) * 16
    pltpu.sync_copy(tab_hbm, idxv)

    slabs = [slab_a, slab_b]
    outs = [ov_a, ov_b]
    gsems = [gs_a, gs_b]
    osems = [os_a, os_b]
    gh = [None, None]
    oh = [None, None]

    g0 = wid * _SLABS
    gh[0] = pltpu.async_copy(x_hbm.at[g0], slabs[0], gsems[0])
    for s in range(_SLABS):
        cur = s & 1
        if s + 1 < _SLABS:
            gh[1 - cur] = pltpu.async_copy(
                x_hbm.at[g0 + s + 1], slabs[1 - cur], gsems[1 - cur])
        gh[cur].wait()
        grp, quarter = s // 4, s % 4
        op = grp & 1
        if quarter == 0 and oh[op] is not None:
            oh[op].wait()
        _compute_slab(slabs[cur], idxv, outs[op], quarter)
        if quarter == 3:
            oh[op] = pltpu.async_copy(
                outs[op], out_hbm.at[b, :, qbase + grp], osems[op])
    oh[0].wait()
    oh[1].wait()


_sc_kernel = functools.partial(
    pl.kernel,
    out_type=jax.ShapeDtypeStruct((_B, _C, 32, 128), jnp.float32),
    mesh=plsc.VectorSubcoreMesh(core_axis_name="c", subcore_axis_name="s"),
    scratch_types=[
        pltpu.VMEM((512, 16), jnp.int32),
        pltpu.VMEM((128, 128), jnp.float32),
        pltpu.VMEM((128, 128), jnp.float32),
        pltpu.VMEM((_C, 128), jnp.float32),
        pltpu.VMEM((_C, 128), jnp.float32),
        pltpu.SemaphoreType.DMA,
        pltpu.SemaphoreType.DMA,
        pltpu.SemaphoreType.DMA,
        pltpu.SemaphoreType.DMA,
    ],
    compiler_params=pltpu.CompilerParams(needs_layout_passes=False),
)(_sc_body)


def kernel(x, rp_map_idx):
    t = jnp.transpose(x, (0, 2, 3, 1))
    t = t.reshape(_B, _H, 8, 8, 4, 128)
    t = jnp.transpose(t, (0, 1, 2, 4, 3, 5))
    x3 = t.reshape(2 * _B * _H, 128, 128)

    rp = rp_map_idx.astype(jnp.int32).reshape(_C * _K)
    lanes = jnp.arange(16, dtype=jnp.int32)
    wpat = (lanes // 8) * 32 + (lanes % 8)
    irow = wpat[None, :] + ((rp // 128) * 8)[:, None]
    icol = jnp.broadcast_to((rp % 128)[:, None], (_C * _K, 16))
    tab = jnp.concatenate([irow, icol], axis=0)

    y5 = _sc_kernel(x3, tab)
    return y5.reshape(_B, _C, _H, _W)


# speedup vs baseline: 1.4553x; 1.2965x over previous
import functools

import jax
import jax.numpy as jnp
from jax import lax
from jax.experimental import pallas as pl
from jax.experimental.pallas import tpu as pltpu
from jax.experimental.pallas import tpu_sc as plsc

_B, _TB, _H, _W = 16, 512, 64, 64
_C, _K = 64, 4
_NW = 32
_SLABS = 64
_THRESH = 2.0


def _compute_slab(slab, idxv, out_v, quarter):
    @plsc.parallel_loop(0, _C, unroll=4)
    def cbody(c):
        accs = [None, None]
        for k in range(_K):
            ir = idxv[4 * c + k, :]
            ic = idxv[256 + 4 * c + k, :]
            for wc in range(2):
                irw = ir + jnp.int32(64 * wc) if wc else ir
                g = plsc.load_gather(slab, [irw, ic])
                accs[wc] = g if k == 0 else accs[wc] + g
        for wc in range(2):
            y = jnp.where(accs[wc] >= _THRESH, jnp.float32(1.0),
                          jnp.float32(0.0))
            out_v[c, pl.ds(quarter * 32 + wc * 16, 16)] = y


def _sc_body(x_hbm, tab_hbm, out_hbm,
             idxv, slab_a, slab_b, ov_a, ov_b,
             gs_a, gs_b, os_a, os_b):
    wid = lax.axis_index("s") * 2 + lax.axis_index("c")
    b = wid // 2
    qbase = (wid % 2) * 16
    pltpu.sync_copy(tab_hbm, idxv)

    slabs = [slab_a, slab_b]
    outs = [ov_a, ov_b]
    gsems = [gs_a, gs_b]
    osems = [os_a, os_b]

    g0 = wid * _SLABS
    pltpu.async_copy(x_hbm.at[g0], slabs[0], gsems[0])
    pltpu.async_copy(x_hbm.at[g0 + 1], slabs[1], gsems[1])

    def outer(it, _):
        base = it * 8
        for j in range(8):
            p = j & 1
            sdyn = base + j
            pltpu.make_async_copy(x_hbm.at[g0], slabs[p], gsems[p]).wait()
            op = (j // 4) & 1
            if j % 4 == 0:
                @pl.when(it > 0)
                def _():
                    pltpu.make_async_copy(
                        outs[op], out_hbm.at[b, :, qbase], osems[op]).wait()
            _compute_slab(slabs[p], idxv, outs[op], j % 4)
            @pl.when(sdyn + 2 < _SLABS)
            def _():
                pltpu.async_copy(
                    x_hbm.at[g0 + sdyn + 2], slabs[p], gsems[p])
            if j % 4 == 3:
                pltpu.async_copy(
                    outs[op], out_hbm.at[b, :, qbase + it * 2 + op],
                    osems[op])
        return 0

    lax.fori_loop(0, _SLABS // 8, outer, 0)
    pltpu.make_async_copy(outs[0], out_hbm.at[b, :, qbase], osems[0]).wait()
    pltpu.make_async_copy(outs[1], out_hbm.at[b, :, qbase], osems[1]).wait()


_sc_kernel = functools.partial(
    pl.kernel,
    out_type=jax.ShapeDtypeStruct((_B, _C, 32, 128), jnp.float32),
    mesh=plsc.VectorSubcoreMesh(core_axis_name="c", subcore_axis_name="s"),
    scratch_types=[
        pltpu.VMEM((512, 16), jnp.int32),
        pltpu.VMEM((128, 128), jnp.float32),
        pltpu.VMEM((128, 128), jnp.float32),
        pltpu.VMEM((_C, 128), jnp.float32),
        pltpu.VMEM((_C, 128), jnp.float32),
        pltpu.SemaphoreType.DMA,
        pltpu.SemaphoreType.DMA,
        pltpu.SemaphoreType.DMA,
        pltpu.SemaphoreType.DMA,
    ],
    compiler_params=pltpu.CompilerParams(needs_layout_passes=False),
)(_sc_body)


def kernel(x, rp_map_idx):
    t = jnp.transpose(x, (0, 2, 3, 1))
    t = t.reshape(_B, _H, 8, 8, 4, 128)
    t = jnp.transpose(t, (0, 1, 2, 4, 3, 5))
    x3 = t.reshape(2 * _B * _H, 128, 128)

    rp = rp_map_idx.astype(jnp.int32).reshape(_C * _K)
    lanes = jnp.arange(16, dtype=jnp.int32)
    wpat = (lanes // 8) * 32 + (lanes % 8)
    irow = wpat[None, :] + ((rp // 128) * 8)[:, None]
    icol = jnp.broadcast_to((rp % 128)[:, None], (_C * _K, 16))
    tab = jnp.concatenate([irow, icol], axis=0)

    y5 = _sc_kernel(x3, tab)
    return y5.reshape(_B, _C, _H, _W)


# speedup vs baseline: 1.4603x; 1.0034x over previous
import functools

import jax
import jax.numpy as jnp
from jax import lax
from jax.experimental import pallas as pl
from jax.experimental.pallas import tpu as pltpu
from jax.experimental.pallas import tpu_sc as plsc

_B, _TB, _H, _W = 16, 512, 64, 64
_C, _K = 64, 4
_NW = 32
_SLABS = 64
_THRESH = 2.0


def _compute_slab(slab, idxv, out_v, quarter):
    @plsc.parallel_loop(0, _C, unroll=8)
    def cbody(c):
        accs = [None, None]
        for k in range(_K):
            ir = idxv[4 * c + k, :]
            ic = idxv[256 + 4 * c + k, :]
            for wc in range(2):
                irw = ir + jnp.int32(64 * wc) if wc else ir
                g = plsc.load_gather(slab, [irw, ic])
                accs[wc] = g if k == 0 else accs[wc] + g
        for wc in range(2):
            y = jnp.where(accs[wc] >= _THRESH, jnp.float32(1.0),
                          jnp.float32(0.0))
            out_v[c, pl.ds(quarter * 32 + wc * 16, 16)] = y


def _sc_body(x_hbm, tab_hbm, out_hbm,
             idxv, slab_a, slab_b, ov_a, ov_b,
             gs_a, gs_b, os_a, os_b):
    wid = lax.axis_index("s") * 2 + lax.axis_index("c")
    b = wid // 2
    qbase = (wid % 2) * 16
    pltpu.sync_copy(tab_hbm, idxv)

    slabs = [slab_a, slab_b]
    outs = [ov_a, ov_b]
    gsems = [gs_a, gs_b]
    osems = [os_a, os_b]

    g0 = wid * _SLABS
    pltpu.async_copy(x_hbm.at[g0], slabs[0], gsems[0])
    pltpu.async_copy(x_hbm.at[g0 + 1], slabs[1], gsems[1])

    def outer(it, _):
        base = it * 8
        for j in range(8):
            p = j & 1
            sdyn = base + j
            pltpu.make_async_copy(x_hbm.at[g0], slabs[p], gsems[p]).wait()
            op = (j // 4) & 1
            if j % 4 == 0:
                @pl.when(it > 0)
                def _():
                    pltpu.make_async_copy(
                        outs[op], out_hbm.at[b, :, qbase], osems[op]).wait()
            _compute_slab(slabs[p], idxv, outs[op], j % 4)
            @pl.when(sdyn + 2 < _SLABS)
            def _():
                pltpu.async_copy(
                    x_hbm.at[g0 + sdyn + 2], slabs[p], gsems[p])
            if j % 4 == 3:
                pltpu.async_copy(
                    outs[op], out_hbm.at[b, :, qbase + it * 2 + op],
                    osems[op])
        return 0

    lax.fori_loop(0, _SLABS // 8, outer, 0)
    pltpu.make_async_copy(outs[0], out_hbm.at[b, :, qbase], osems[0]).wait()
    pltpu.make_async_copy(outs[1], out_hbm.at[b, :, qbase], osems[1]).wait()


_sc_kernel = functools.partial(
    pl.kernel,
    out_type=jax.ShapeDtypeStruct((_B, _C, 32, 128), jnp.float32),
    mesh=plsc.VectorSubcoreMesh(core_axis_name="c", subcore_axis_name="s"),
    scratch_types=[
        pltpu.VMEM((512, 16), jnp.int32),
        pltpu.VMEM((128, 128), jnp.float32),
        pltpu.VMEM((128, 128), jnp.float32),
        pltpu.VMEM((_C, 128), jnp.float32),
        pltpu.VMEM((_C, 128), jnp.float32),
        pltpu.SemaphoreType.DMA,
        pltpu.SemaphoreType.DMA,
        pltpu.SemaphoreType.DMA,
        pltpu.SemaphoreType.DMA,
    ],
    compiler_params=pltpu.CompilerParams(needs_layout_passes=False),
)(_sc_body)


def kernel(x, rp_map_idx):
    t = jnp.transpose(x, (0, 2, 3, 1))
    t = t.reshape(_B, _H, 8, 8, 4, 128)
    t = jnp.transpose(t, (0, 1, 2, 4, 3, 5))
    x3 = t.reshape(2 * _B * _H, 128, 128)

    rp = rp_map_idx.astype(jnp.int32).reshape(_C * _K)
    lanes = jnp.arange(16, dtype=jnp.int32)
    wpat = (lanes // 8) * 32 + (lanes % 8)
    irow = wpat[None, :] + ((rp // 128) * 8)[:, None]
    icol = jnp.broadcast_to((rp % 128)[:, None], (_C * _K, 16))
    tab = jnp.concatenate([irow, icol], axis=0)

    y5 = _sc_kernel(x3, tab)
    return y5.reshape(_B, _C, _H, _W)


# speedup vs baseline: 3.5903x; 2.4586x over previous
import functools

import jax
import jax.numpy as jnp
from jax import lax
from jax.experimental import pallas as pl
from jax.experimental.pallas import tpu as pltpu
from jax.experimental.pallas import tpu_sc as plsc

_B, _TB, _H, _W = 16, 512, 64, 64
_C, _K = 64, 4
_NW = 32
_SLABS = 64
_THRESH = 2.0


def _compute_slab(slab, tabv, out_v, quarter):
    rtabs = [tabv[r, :] for r in range(16)]
    ctabs = [tabv[16 + r, :] for r in range(16)]

    @plsc.parallel_loop(0, 32, unroll=4)
    def pbody(p):
        rowb = (p >> 3) * 32 + (p & 7)
        for j in range(4):
            acc = None
            for k in range(_K):
                ir = rtabs[4 * j + k] + rowb
                g = plsc.load_gather(slab, [ir, ctabs[4 * j + k]])
                acc = g if k == 0 else acc + g
            y = jnp.where(acc >= _THRESH, jnp.float32(1.0), jnp.float32(0.0))
            out_v[quarter * 32 + p, pl.ds(16 * j, 16)] = y


def _sc_body(x_hbm, tab_hbm, out_hbm,
             tabv, slab_a, slab_b, ov_a, ov_b,
             gs_a, gs_b, os_a, os_b):
    wid = lax.axis_index("s") * 2 + lax.axis_index("c")
    b = wid // 2
    qbase = (wid % 2) * 16
    pltpu.sync_copy(tab_hbm, tabv)

    slabs = [slab_a, slab_b]
    outs = [ov_a, ov_b]
    gsems = [gs_a, gs_b]
    osems = [os_a, os_b]

    g0 = wid * _SLABS
    pltpu.async_copy(x_hbm.at[g0], slabs[0], gsems[0])
    pltpu.async_copy(x_hbm.at[g0 + 1], slabs[1], gsems[1])

    def outer(it, _):
        base = it * 8
        for j in range(8):
            p = j & 1
            sdyn = base + j
            pltpu.make_async_copy(x_hbm.at[g0], slabs[p], gsems[p]).wait()
            op = (j // 4) & 1
            if j % 4 == 0:
                @pl.when(it > 0)
                def _():
                    pltpu.make_async_copy(
                        outs[op], out_hbm.at[b, qbase], osems[op]).wait()
            _compute_slab(slabs[p], tabv, outs[op], j % 4)
            @pl.when(sdyn + 2 < _SLABS)
            def _():
                pltpu.async_copy(
                    x_hbm.at[g0 + sdyn + 2], slabs[p], gsems[p])
            if j % 4 == 3:
                pltpu.async_copy(
                    outs[op], out_hbm.at[b, qbase + it * 2 + op],
                    osems[op])
        return 0

    lax.fori_loop(0, _SLABS // 8, outer, 0)
    pltpu.make_async_copy(outs[0], out_hbm.at[b, qbase], osems[0]).wait()
    pltpu.make_async_copy(outs[1], out_hbm.at[b, qbase], osems[1]).wait()


_sc_kernel = functools.partial(
    pl.kernel,
    out_type=jax.ShapeDtypeStruct((_B, 32, 128, _C), jnp.float32),
    mesh=plsc.VectorSubcoreMesh(core_axis_name="c", subcore_axis_name="s"),
    scratch_types=[
        pltpu.VMEM((32, 16), jnp.int32),
        pltpu.VMEM((128, 128), jnp.float32),
        pltpu.VMEM((128, 128), jnp.float32),
        pltpu.VMEM((128, _C), jnp.float32),
        pltpu.VMEM((128, _C), jnp.float32),
        pltpu.SemaphoreType.DMA,
        pltpu.SemaphoreType.DMA,
        pltpu.SemaphoreType.DMA,
        pltpu.SemaphoreType.DMA,
    ],
    compiler_params=pltpu.CompilerParams(needs_layout_passes=False),
)(_sc_body)


def kernel(x, rp_map_idx):
    t = jnp.transpose(x, (0, 2, 3, 1))
    t = t.reshape(_B, _H, 8, 8, 4, 128)
    t = jnp.transpose(t, (0, 1, 2, 4, 3, 5))
    x3 = t.reshape(2 * _B * _H, 128, 128)

    rp = rp_map_idx.astype(jnp.int32)
    rowoff = (rp // 128) * 8
    coloff = rp % 128
    rtab = jnp.transpose(rowoff.reshape(4, 16, _K), (0, 2, 1)).reshape(16, 16)
    ctab = jnp.transpose(coloff.reshape(4, 16, _K), (0, 2, 1)).reshape(16, 16)
    tab = jnp.concatenate([rtab, ctab], axis=0)

    y6 = _sc_kernel(x3, tab)
    y = y6.reshape(_B, 32, 2, _W, _C)
    y = jnp.transpose(y, (0, 4, 1, 2, 3))
    return y.reshape(_B, _C, _H, _W)


# speedup vs baseline: 4.0541x; 1.1292x over previous
import functools

import jax
import jax.numpy as jnp
from jax import lax
from jax.experimental import pallas as pl
from jax.experimental.pallas import tpu as pltpu
from jax.experimental.pallas import tpu_sc as plsc

_B, _TB, _H, _W = 16, 512, 64, 64
_C, _K = 64, 4
_NW = 32
_SLABS = 64
_THRESH = 2.0


def _compute_slab(slab, tabv, out_v, quarter):
    rtabs = [tabv[r, :] for r in range(16)]
    ctabs = [tabv[16 + r, :] for r in range(16)]

    @plsc.parallel_loop(0, 32, unroll=4)
    def pbody(p):
        rowb = (p >> 3) * 32 + (p & 7)
        for j in range(4):
            acc = None
            for k in range(_K):
                ir = rtabs[4 * j + k] + rowb
                g = plsc.load_gather(slab, [ir, ctabs[4 * j + k]])
                acc = g if k == 0 else acc + g
            y = jnp.where(acc >= _THRESH, jnp.float32(1.0), jnp.float32(0.0))
            out_v[quarter * 32 + p, pl.ds(16 * j, 16)] = y


def _sc_body(x_hbm, tab_hbm, out_hbm,
             tabv, slab_a, slab_b, slab_c, slab_d, ov_a, ov_b,
             gs_a, gs_b, gs_c, gs_d, os_a, os_b):
    wid = lax.axis_index("s") * 2 + lax.axis_index("c")
    b = wid // 2
    qbase = (wid % 2) * 16
    pltpu.sync_copy(tab_hbm, tabv)

    slabs = [slab_a, slab_b, slab_c, slab_d]
    outs = [ov_a, ov_b]
    gsems = [gs_a, gs_b, gs_c, gs_d]
    osems = [os_a, os_b]

    g0 = wid * _SLABS
    for i in range(4):
        pltpu.async_copy(x_hbm.at[g0 + i], slabs[i], gsems[i])

    def outer(it, _):
        base = it * 8
        for j in range(8):
            p = j & 3
            sdyn = base + j
            pltpu.make_async_copy(x_hbm.at[g0], slabs[p], gsems[p]).wait()
            op = (j // 4) & 1
            if j % 4 == 0:
                @pl.when(it > 0)
                def _():
                    pltpu.make_async_copy(
                        outs[op], out_hbm.at[b, qbase], osems[op]).wait()
            _compute_slab(slabs[p], tabv, outs[op], j % 4)
            @pl.when(sdyn + 4 < _SLABS)
            def _():
                pltpu.async_copy(
                    x_hbm.at[g0 + sdyn + 4], slabs[p], gsems[p])
            if j % 4 == 3:
                pltpu.async_copy(
                    outs[op], out_hbm.at[b, qbase + it * 2 + op],
                    osems[op])
        return 0

    lax.fori_loop(0, _SLABS // 8, outer, 0)
    pltpu.make_async_copy(outs[0], out_hbm.at[b, qbase], osems[0]).wait()
    pltpu.make_async_copy(outs[1], out_hbm.at[b, qbase], osems[1]).wait()


_sc_kernel = functools.partial(
    pl.kernel,
    out_type=jax.ShapeDtypeStruct((_B, 32, 128, _C), jnp.float32),
    mesh=plsc.VectorSubcoreMesh(core_axis_name="c", subcore_axis_name="s"),
    scratch_types=[
        pltpu.VMEM((32, 16), jnp.int32),
        pltpu.VMEM((128, 128), jnp.float32),
        pltpu.VMEM((128, 128), jnp.float32),
        pltpu.VMEM((128, 128), jnp.float32),
        pltpu.VMEM((128, 128), jnp.float32),
        pltpu.VMEM((128, _C), jnp.float32),
        pltpu.VMEM((128, _C), jnp.float32),
        pltpu.SemaphoreType.DMA,
        pltpu.SemaphoreType.DMA,
        pltpu.SemaphoreType.DMA,
        pltpu.SemaphoreType.DMA,
        pltpu.SemaphoreType.DMA,
        pltpu.SemaphoreType.DMA,
    ],
    compiler_params=pltpu.CompilerParams(needs_layout_passes=False),
)(_sc_body)


def kernel(x, rp_map_idx):
    t = jnp.transpose(x, (0, 2, 3, 1))
    t = t.reshape(_B, _H, 8, 8, 4, 128)
    t = jnp.transpose(t, (0, 1, 2, 4, 3, 5))
    x3 = t.reshape(2 * _B * _H, 128, 128)

    rp = rp_map_idx.astype(jnp.int32)
    rowoff = (rp // 128) * 8
    coloff = rp % 128
    rtab = jnp.transpose(rowoff.reshape(4, 16, _K), (0, 2, 1)).reshape(16, 16)
    ctab = jnp.transpose(coloff.reshape(4, 16, _K), (0, 2, 1)).reshape(16, 16)
    tab = jnp.concatenate([rtab, ctab], axis=0)

    y6 = _sc_kernel(x3, tab)
    y = y6.reshape(_B, 32, 2, _W, _C)
    y = jnp.transpose(y, (0, 4, 1, 2, 3))
    return y.reshape(_B, _C, _H, _W)
